# Initial kernel scaffold; baseline (speedup 1.0000x reference)
import jax
import jax.numpy as jnp
from jax.experimental import pallas as pl


def kernel(in0, in1, ochlv):
    raise NotImplementedError("write your pallas kernel here")


# speedup vs baseline: 1.1354x; 1.1354x over previous
import jax
import jax.numpy as jnp
from jax import lax
from jax.experimental import pallas as pl
from jax.experimental.pallas import tpu as pltpu

_PRCCD = 3


def _dot(m, x, dims=(((1,), (0,)), ((), ()))):
    return lax.dot_general(m, x, dims, precision=lax.Precision.HIGHEST,
                           preferred_element_type=jnp.float32)


def _arranger_body(closes_ref, oc_ref, a_ref, b_ref, o0_ref, o1_ref, o2_ref, ord_ref):
    T, S = closes_ref.shape[1], closes_ref.shape[2]
    closes = closes_ref[0]

    cond = closes != 0.0
    iota_s = lax.broadcasted_iota(jnp.int32, (T, S), 1)
    masked = jnp.where(cond, iota_s, S)
    fi = jnp.min(masked, axis=1, keepdims=True)
    onehot = iota_s == fi
    starts = jnp.sum(jnp.where(onehot, closes, 0.0), axis=1, keepdims=True)
    last = closes[:, S - 1:S]
    p_col = jnp.where(starts != 0.0, (last - starts) / starts,
                      jnp.zeros_like(starts))

    iota_j = lax.broadcasted_iota(jnp.int32, (T, T), 0)
    iota_i = lax.broadcasted_iota(jnp.int32, (T, T), 1)
    ident = (iota_j == iota_i).astype(jnp.float32)

    p_row = _dot(p_col, ident, (((0,), (0,)), ((), ())))

    pj = jnp.broadcast_to(p_col, (T, T))
    pi = jnp.broadcast_to(p_row, (T, T))
    beats = (pj > pi) | ((pj == pi) & (iota_j < iota_i))
    rank = jnp.sum(beats.astype(jnp.float32), axis=0, keepdims=True)

    P = (jnp.broadcast_to(rank.astype(jnp.int32), (T, T)) == iota_j).astype(jnp.float32)

    iota_col = lax.broadcasted_iota(jnp.int32, (T, 1), 0).astype(jnp.float32)
    ord_ref[0] = _dot(P, iota_col).astype(jnp.int32)

    o0_ref[0] = _dot(P, a_ref[0])
    o1_ref[0] = _dot(P, b_ref[0])
    o2_ref[0] = _dot(P, oc_ref[0])


def kernel(in0, in1, ochlv):
    B, T, S, F = ochlv.shape
    D = S * F
    closes = ochlv[..., _PRCCD]
    oc2 = ochlv.reshape(B, T, D)

    bmap = lambda b: (b, 0, 0)
    out0, out1, out2f, orders3 = pl.pallas_call(
        _arranger_body,
        grid=(B,),
        in_specs=[
            pl.BlockSpec((1, T, S), bmap),
            pl.BlockSpec((1, T, D), bmap),
            pl.BlockSpec((1, T, in0.shape[2]), bmap),
            pl.BlockSpec((1, T, in1.shape[2]), bmap),
        ],
        out_specs=[
            pl.BlockSpec((1, T, in0.shape[2]), bmap),
            pl.BlockSpec((1, T, in1.shape[2]), bmap),
            pl.BlockSpec((1, T, D), bmap),
            pl.BlockSpec((1, T, 1), bmap),
        ],
        out_shape=[
            jax.ShapeDtypeStruct((B, T, in0.shape[2]), in0.dtype),
            jax.ShapeDtypeStruct((B, T, in1.shape[2]), in1.dtype),
            jax.ShapeDtypeStruct((B, T, D), ochlv.dtype),
            jax.ShapeDtypeStruct((B, T, 1), jnp.int32),
        ],
        compiler_params=pltpu.CompilerParams(
            dimension_semantics=("parallel",),
        ),
    )(closes, oc2, in0, in1)

    out2 = out2f.reshape(B, T, S, F)
    orders = orders3.reshape(B, T)
    return ((out0, out1, out2), orders)
